# baseline (speedup 1.0000x reference)
import jax
import jax.numpy as jnp
from jax.experimental import pallas as pl
from jax.experimental.pallas import tpu as pltpu

_MAX_STEPS = 3
_N_EXP = 8
_D_MODEL = 768
_D_FF = 3072
_N_TOK = 2048

_ROW_BLK = 256
_N_ROW = _N_TOK // _ROW_BLK
_FF_BLK = 512
_N_FF = _D_FF // _FF_BLK


def _preproc_body(enc_ref, wc_ref, ws_ref, x_ref, psum_ref):
    i = pl.program_id(0)
    t = jnp.tanh(jnp.dot(enc_ref[...], wc_ref[...], preferred_element_type=jnp.float32))
    x = jnp.dot(t, ws_ref[...], preferred_element_type=jnp.float32)
    x_ref[...] = x

    @pl.when(i == 0)
    def _():
        psum_ref[...] = jnp.zeros_like(psum_ref)

    psum_ref[...] += jnp.sum(x, axis=0, keepdims=True)


def _route_body(psum_ref, wr_ref, keys_ref, chosen_ref, gate_ref):
    pooled = psum_ref[...] * (1.0 / _N_TOK)
    rv = jnp.tanh(jnp.dot(pooled, wr_ref[...], preferred_element_type=jnp.float32))
    scores = jax.lax.dot_general(
        rv, keys_ref[...], (((1,), (1,)), ((), ())),
        preferred_element_type=jnp.float32)
    m = jnp.max(scores)
    idx = jax.lax.broadcasted_iota(jnp.int32, (1, _N_EXP), 1)
    chosen = jnp.min(jnp.where(scores == m, idx, _N_EXP))
    e = jnp.exp(scores - m)
    gate = jnp.sum(jnp.where(idx == chosen, e, 0.0)) / jnp.sum(e)
    chosen_ref[0, 0] = chosen
    gate_ref[0, 0] = gate


def _ffn_body(chosen_sref, gate_sref, x_ref, w1_ref, w2_ref, out_ref, psum_ref):
    f = pl.program_id(0)
    h = jax.nn.gelu(jnp.dot(x_ref[...], w1_ref[0], preferred_element_type=jnp.float32))
    contrib = jnp.dot(h, w2_ref[0], preferred_element_type=jnp.float32)

    @pl.when(f == 0)
    def _():
        out_ref[...] = contrib

    @pl.when(f > 0)
    def _():
        out_ref[...] += contrib

    @pl.when(f == _N_FF - 1)
    def _():
        o = out_ref[...] * gate_sref[0]
        out_ref[...] = o
        psum_ref[...] = jnp.sum(o, axis=0, keepdims=True)


def _preproc(encodings, W_combine, W_state):
    return pl.pallas_call(
        _preproc_body,
        grid=(_N_ROW,),
        in_specs=[
            pl.BlockSpec((_ROW_BLK, _D_MODEL), lambda i: (i, 0)),
            pl.BlockSpec((_D_MODEL, _D_MODEL), lambda i: (0, 0)),
            pl.BlockSpec((_D_MODEL, _D_MODEL), lambda i: (0, 0)),
        ],
        out_specs=[
            pl.BlockSpec((_ROW_BLK, _D_MODEL), lambda i: (i, 0)),
            pl.BlockSpec((1, _D_MODEL), lambda i: (0, 0)),
        ],
        out_shape=[
            jax.ShapeDtypeStruct((_N_TOK, _D_MODEL), jnp.float32),
            jax.ShapeDtypeStruct((1, _D_MODEL), jnp.float32),
        ],
        compiler_params=pltpu.CompilerParams(dimension_semantics=("arbitrary",)),
    )(encodings, W_combine, W_state)


def _route(psum, W_router, expert_keys):
    return pl.pallas_call(
        _route_body,
        in_specs=[
            pl.BlockSpec(memory_space=pltpu.VMEM),
            pl.BlockSpec(memory_space=pltpu.VMEM),
            pl.BlockSpec(memory_space=pltpu.VMEM),
        ],
        out_specs=[
            pl.BlockSpec(memory_space=pltpu.SMEM),
            pl.BlockSpec(memory_space=pltpu.SMEM),
        ],
        out_shape=[
            jax.ShapeDtypeStruct((1, 1), jnp.int32),
            jax.ShapeDtypeStruct((1, 1), jnp.float32),
        ],
    )(psum, W_router, expert_keys)


def _ffn(chosen, gate, x, expert_W1, expert_W2):
    grid_spec = pltpu.PrefetchScalarGridSpec(
        num_scalar_prefetch=2,
        grid=(_N_FF,),
        in_specs=[
            pl.BlockSpec((_N_TOK, _D_MODEL), lambda f, c, g: (0, 0)),
            pl.BlockSpec((1, _D_MODEL, _FF_BLK), lambda f, c, g: (c[0], 0, f)),
            pl.BlockSpec((1, _FF_BLK, _D_MODEL), lambda f, c, g: (c[0], f, 0)),
        ],
        out_specs=[
            pl.BlockSpec((_N_TOK, _D_MODEL), lambda f, c, g: (0, 0)),
            pl.BlockSpec((1, _D_MODEL), lambda f, c, g: (0, 0)),
        ],
    )
    return pl.pallas_call(
        _ffn_body,
        grid_spec=grid_spec,
        out_shape=[
            jax.ShapeDtypeStruct((_N_TOK, _D_MODEL), jnp.float32),
            jax.ShapeDtypeStruct((1, _D_MODEL), jnp.float32),
        ],
        compiler_params=pltpu.CompilerParams(dimension_semantics=("arbitrary",)),
    )(chosen, gate, x, expert_W1, expert_W2)


def kernel(encodings, W_combine, W_router, W_state, expert_keys, expert_W1, expert_W2):
    x0, psum0 = _preproc(encodings, W_combine, W_state)

    def step(x, psum):
        chosen2d, gate2d = _route(psum, W_router, expert_keys)
        out, psum_new = _ffn(chosen2d.reshape((1,)), gate2d.reshape((1,)),
                             x, expert_W1, expert_W2)
        return out, psum_new, chosen2d[0, 0] == 0

    x, psum, done = step(x0, psum0)
    for _ in range(_MAX_STEPS - 1):
        x, psum, done = jax.lax.cond(
            done,
            lambda x, psum: (x, psum, jnp.bool_(True)),
            lambda x, psum: step(x, psum),
            x, psum)
    return x


# speedup vs baseline: 1.1013x; 1.1013x over previous
import jax
import jax.numpy as jnp
from jax.experimental import pallas as pl
from jax.experimental.pallas import tpu as pltpu

_MAX_STEPS = 3
_N_EXP = 8
_D_MODEL = 768
_D_FF = 3072
_N_TOK = 2048

_ROW_BLK = 256
_N_ROW = _N_TOK // _ROW_BLK


def _route(psum_ref, wr_ref, keys_ref, chosen_ref, gate_ref):
    pooled = psum_ref[...] * (1.0 / _N_TOK)
    rv = jnp.tanh(jnp.dot(pooled, wr_ref[...], preferred_element_type=jnp.float32))
    scores = jax.lax.dot_general(
        rv, keys_ref[...], (((1,), (1,)), ((), ())),
        preferred_element_type=jnp.float32)
    m = jnp.max(scores)
    idx = jax.lax.broadcasted_iota(jnp.int32, (1, _N_EXP), 1)
    chosen = jnp.min(jnp.where(scores == m, idx, _N_EXP))
    e = jnp.exp(scores - m)
    gate = jnp.sum(jnp.where(idx == chosen, e, 0.0)) / jnp.sum(e)
    chosen_ref[0, 0] = chosen
    gate_ref[0, 0] = gate


def _preproc_body(enc_ref, wc_ref, ws_ref, wr_ref, keys_ref,
                  x_ref, chosen_ref, gate_ref, psum_ref):
    i = pl.program_id(0)
    t = jnp.tanh(jnp.dot(enc_ref[...], wc_ref[...], preferred_element_type=jnp.float32))
    x = jnp.dot(t, ws_ref[...], preferred_element_type=jnp.float32)
    x_ref[...] = x

    @pl.when(i == 0)
    def _():
        psum_ref[...] = jnp.zeros_like(psum_ref)

    psum_ref[...] += jnp.sum(x, axis=0, keepdims=True)

    @pl.when(i == _N_ROW - 1)
    def _():
        _route(psum_ref, wr_ref, keys_ref, chosen_ref, gate_ref)


def _ffn_body(chosen_sref, gate_sref, x_ref, w1_ref, w2_ref, wr_ref, keys_ref,
              out_ref, chosen_ref, gate_ref, psum_ref):
    r = pl.program_id(0)
    h = jax.nn.gelu(jnp.dot(x_ref[...], w1_ref[0], preferred_element_type=jnp.float32))
    o = jnp.dot(h, w2_ref[0], preferred_element_type=jnp.float32) * gate_sref[0]
    out_ref[...] = o

    @pl.when(r == 0)
    def _():
        psum_ref[...] = jnp.zeros_like(psum_ref)

    psum_ref[...] += jnp.sum(o, axis=0, keepdims=True)

    @pl.when(r == _N_ROW - 1)
    def _():
        _route(psum_ref, wr_ref, keys_ref, chosen_ref, gate_ref)


def _preproc(encodings, W_combine, W_state, W_router, expert_keys):
    return pl.pallas_call(
        _preproc_body,
        grid=(_N_ROW,),
        in_specs=[
            pl.BlockSpec((_ROW_BLK, _D_MODEL), lambda i: (i, 0)),
            pl.BlockSpec((_D_MODEL, _D_MODEL), lambda i: (0, 0)),
            pl.BlockSpec((_D_MODEL, _D_MODEL), lambda i: (0, 0)),
            pl.BlockSpec((_D_MODEL, _D_MODEL), lambda i: (0, 0)),
            pl.BlockSpec((_N_EXP, _D_MODEL), lambda i: (0, 0)),
        ],
        out_specs=[
            pl.BlockSpec((_ROW_BLK, _D_MODEL), lambda i: (i, 0)),
            pl.BlockSpec((1, 1), lambda i: (0, 0), memory_space=pltpu.SMEM),
            pl.BlockSpec((1, 1), lambda i: (0, 0), memory_space=pltpu.SMEM),
        ],
        out_shape=[
            jax.ShapeDtypeStruct((_N_TOK, _D_MODEL), jnp.float32),
            jax.ShapeDtypeStruct((1, 1), jnp.int32),
            jax.ShapeDtypeStruct((1, 1), jnp.float32),
        ],
        scratch_shapes=[pltpu.VMEM((1, _D_MODEL), jnp.float32)],
        compiler_params=pltpu.CompilerParams(dimension_semantics=("arbitrary",)),
    )(encodings, W_combine, W_state, W_router, expert_keys)


def _ffn(chosen, gate, x, expert_W1, expert_W2, W_router, expert_keys):
    grid_spec = pltpu.PrefetchScalarGridSpec(
        num_scalar_prefetch=2,
        grid=(_N_ROW,),
        in_specs=[
            pl.BlockSpec((_ROW_BLK, _D_MODEL), lambda r, c, g: (r, 0)),
            pl.BlockSpec((1, _D_MODEL, _D_FF), lambda r, c, g: (c[0], 0, 0)),
            pl.BlockSpec((1, _D_FF, _D_MODEL), lambda r, c, g: (c[0], 0, 0)),
            pl.BlockSpec((_D_MODEL, _D_MODEL), lambda r, c, g: (0, 0)),
            pl.BlockSpec((_N_EXP, _D_MODEL), lambda r, c, g: (0, 0)),
        ],
        out_specs=[
            pl.BlockSpec((_ROW_BLK, _D_MODEL), lambda r, c, g: (r, 0)),
            pl.BlockSpec((1, 1), lambda r, c, g: (0, 0), memory_space=pltpu.SMEM),
            pl.BlockSpec((1, 1), lambda r, c, g: (0, 0), memory_space=pltpu.SMEM),
        ],
        scratch_shapes=[pltpu.VMEM((1, _D_MODEL), jnp.float32)],
    )
    return pl.pallas_call(
        _ffn_body,
        grid_spec=grid_spec,
        out_shape=[
            jax.ShapeDtypeStruct((_N_TOK, _D_MODEL), jnp.float32),
            jax.ShapeDtypeStruct((1, 1), jnp.int32),
            jax.ShapeDtypeStruct((1, 1), jnp.float32),
        ],
        compiler_params=pltpu.CompilerParams(dimension_semantics=("arbitrary",)),
    )(chosen, gate, x, expert_W1, expert_W2, W_router, expert_keys)


def kernel(encodings, W_combine, W_router, W_state, expert_keys, expert_W1, expert_W2):
    x0, chosen, gate = _preproc(encodings, W_combine, W_state, W_router, expert_keys)

    def step(x, chosen2d, gate2d):
        out, nchosen, ngate = _ffn(chosen2d.reshape((1,)), gate2d.reshape((1,)),
                                   x, expert_W1, expert_W2, W_router, expert_keys)
        return out, nchosen, ngate, chosen2d[0, 0] == 0

    x, chosen, gate, done = step(x0, chosen, gate)
    for _ in range(_MAX_STEPS - 1):
        x, chosen, gate, done = jax.lax.cond(
            done,
            lambda x, c, g: (x, c, g, jnp.bool_(True)),
            lambda x, c, g: step(x, c, g),
            x, chosen, gate)
    return x


# speedup vs baseline: 1.2621x; 1.1461x over previous
import jax
import jax.numpy as jnp
from jax.experimental import pallas as pl
from jax.experimental.pallas import tpu as pltpu

_MAX_STEPS = 3
_N_EXP = 8
_D_MODEL = 768
_D_FF = 3072
_N_TOK = 2048

_ROW_BLK = 256
_N_ROW = _N_TOK // _ROW_BLK


def _route(psum_ref, wr_ref, keys_ref, chosen_ref, gate_ref):
    pooled = psum_ref[...] * (1.0 / _N_TOK)
    rv = jnp.tanh(jnp.dot(pooled, wr_ref[...], preferred_element_type=jnp.float32))
    scores = jax.lax.dot_general(
        rv, keys_ref[...], (((1,), (1,)), ((), ())),
        preferred_element_type=jnp.float32)
    m = jnp.max(scores)
    idx = jax.lax.broadcasted_iota(jnp.int32, (1, _N_EXP), 1)
    chosen = jnp.min(jnp.where(scores == m, idx, _N_EXP))
    e = jnp.exp(scores - m)
    gate = jnp.sum(jnp.where(idx == chosen, e, 0.0)) / jnp.sum(e)
    chosen_ref[0, 0] = chosen
    gate_ref[0, 0] = gate


def _mega_body(enc_ref, wc_ref, ws_ref, wr_ref, keys_ref, ew1_ref, ew2_ref,
               state_ref, w1_v, w2_v, psum_ref, chosen_ref, gate_ref, done_ref,
               w1_sem, w2_sem):

    psum_ref[...] = jnp.zeros_like(psum_ref)
    done_ref[0, 0] = 0

    def pre_tile(r, carry):
        rows = pl.ds(r * _ROW_BLK, _ROW_BLK)
        t = jnp.tanh(jnp.dot(enc_ref[rows, :], wc_ref[...],
                             preferred_element_type=jnp.float32))
        x = jnp.dot(t, ws_ref[...], preferred_element_type=jnp.float32)
        state_ref[rows, :] = x
        psum_ref[...] += jnp.sum(x, axis=0, keepdims=True)
        return carry

    jax.lax.fori_loop(0, _N_ROW, pre_tile, 0)
    _route(psum_ref, wr_ref, keys_ref, chosen_ref, gate_ref)

    def emit_step():
        c = chosen_ref[0, 0]
        g = gate_ref[0, 0]
        w1_copy = pltpu.make_async_copy(ew1_ref.at[c], w1_v, w1_sem)
        w2_copy = pltpu.make_async_copy(ew2_ref.at[c], w2_v, w2_sem)
        w1_copy.start()
        w2_copy.start()
        psum_ref[...] = jnp.zeros_like(psum_ref)
        w1_copy.wait()

        rows0 = pl.ds(0, _ROW_BLK)
        h0 = jax.nn.gelu(jnp.dot(state_ref[rows0, :], w1_v[...],
                                 preferred_element_type=jnp.float32))
        w2_copy.wait()
        o0 = jnp.dot(h0, w2_v[...], preferred_element_type=jnp.float32) * g
        state_ref[rows0, :] = o0
        psum_ref[...] += jnp.sum(o0, axis=0, keepdims=True)

        def ffn_tile(r, carry):
            rows = pl.ds(r * _ROW_BLK, _ROW_BLK)
            h = jax.nn.gelu(jnp.dot(state_ref[rows, :], w1_v[...],
                                    preferred_element_type=jnp.float32))
            o = jnp.dot(h, w2_v[...], preferred_element_type=jnp.float32) * g
            state_ref[rows, :] = o
            psum_ref[...] += jnp.sum(o, axis=0, keepdims=True)
            return carry

        jax.lax.fori_loop(1, _N_ROW, ffn_tile, 0)

        @pl.when(c == 0)
        def _():
            done_ref[0, 0] = 1

        _route(psum_ref, wr_ref, keys_ref, chosen_ref, gate_ref)

    emit_step()
    for _ in range(_MAX_STEPS - 1):
        @pl.when(done_ref[0, 0] == 0)
        def _():
            emit_step()


def kernel(encodings, W_combine, W_router, W_state, expert_keys, expert_W1, expert_W2):
    return pl.pallas_call(
        _mega_body,
        in_specs=[
            pl.BlockSpec(memory_space=pltpu.VMEM),
            pl.BlockSpec(memory_space=pltpu.VMEM),
            pl.BlockSpec(memory_space=pltpu.VMEM),
            pl.BlockSpec(memory_space=pltpu.VMEM),
            pl.BlockSpec(memory_space=pltpu.VMEM),
            pl.BlockSpec(memory_space=pltpu.MemorySpace.HBM),
            pl.BlockSpec(memory_space=pltpu.MemorySpace.HBM),
        ],
        out_specs=pl.BlockSpec(memory_space=pltpu.VMEM),
        out_shape=jax.ShapeDtypeStruct((_N_TOK, _D_MODEL), jnp.float32),
        scratch_shapes=[
            pltpu.VMEM((_D_MODEL, _D_FF), jnp.float32),
            pltpu.VMEM((_D_FF, _D_MODEL), jnp.float32),
            pltpu.VMEM((1, _D_MODEL), jnp.float32),
            pltpu.SMEM((1, 1), jnp.int32),
            pltpu.SMEM((1, 1), jnp.float32),
            pltpu.SMEM((1, 1), jnp.int32),
            pltpu.SemaphoreType.DMA,
            pltpu.SemaphoreType.DMA,
        ],
    )(encodings, W_combine, W_state, W_router, expert_keys, expert_W1, expert_W2)
